# baseline (speedup 1.0000x reference)
import jax
import jax.numpy as jnp
from jax import lax
from jax.experimental import pallas as pl
from jax.experimental.pallas import tpu as pltpu
from jax.experimental.pallas import tpu_sc as plsc

B = 16
MAX_NODES = 2048
NUM_RULES = 512
NUM_COND = 8192
FLAT = MAX_NODES * NUM_RULES

NC = 2
NS = 16
NW = NC * NS
E_PER_W = FLAT // NW
NODES_PER_W = E_PER_W // NUM_RULES
ECHUNK = 1024
NODES_PER_CHUNK = ECHUNK // NUM_RULES
PAIRS = NODES_PER_CHUNK * B
NCHUNK = E_PER_W // ECHUNK
LANES = 16
NSLICE = NUM_RULES // LANES
NEG = jnp.float32(-100000.0)


def _sc_body(cond_hbm, maskx_hbm, priors_hbm, unc_hbm, table_hbm, out_hbm,
             idx_v, mskx_v, unc_v, rows_v, prior_v, out_v,
             gsem0, gsem1, psem0, psem1, osem0, osem1):
    wid = lax.axis_index("s") * NC + lax.axis_index("c")
    e0 = wid * E_PER_W
    gsem = (gsem0, gsem1)
    psem = (psem0, psem1)
    osem = (osem0, osem1)

    pltpu.sync_copy(cond_hbm.at[pl.ds(wid * NODES_PER_W * B, NODES_PER_W * B)],
                    idx_v)
    pltpu.sync_copy(
        maskx_hbm.at[pl.ds(wid * NODES_PER_W * B * LANES,
                           NODES_PER_W * B * LANES)], mskx_v)
    pltpu.sync_copy(unc_hbm, unc_v)

    def issue_in(c, b):
        pltpu.async_copy(table_hbm.at[idx_v.at[pl.ds(c * PAIRS, PAIRS)]],
                         rows_v.at[b], gsem[b])
        pltpu.async_copy(
            priors_hbm.at[pl.ds(0, B), pl.ds(e0 + c * ECHUNK, ECHUNK)],
            prior_v.at[b], psem[b])

    def wait_in(c, b):
        pltpu.make_async_copy(table_hbm.at[idx_v.at[pl.ds(c * PAIRS, PAIRS)]],
                              rows_v.at[b], gsem[b]).wait()
        pltpu.make_async_copy(
            priors_hbm.at[pl.ds(0, B), pl.ds(e0 + c * ECHUNK, ECHUNK)],
            prior_v.at[b], psem[b]).wait()

    def out_copy(c, b):
        return pltpu.make_async_copy(
            out_v.at[b],
            out_hbm.at[pl.ds(0, B), pl.ds(e0 + c * ECHUNK, ECHUNK)], osem[b])

    def compute_chunk(c, buf):
        rows = rows_v.at[buf]
        prior = prior_v.at[buf]
        out = out_v.at[buf]
        for nl in range(NODES_PER_CHUNK):

            @plsc.parallel_loop(0, B, unroll=4)
            def pair_body(bb):
                p = nl * B + bb
                mv = mskx_v[pl.ds((c * PAIRS + p) * LANES, LANES)]
                moff = (jnp.float32(1.0) - mv) * NEG
                for j in range(NSLICE):
                    sl = pl.ds(j * LANES, LANES)
                    osl = pl.ds(nl * NUM_RULES + j * LANES, LANES)
                    out[bb, osl] = (mv * (unc_v[sl] + rows[p, sl]) + moff
                                    + prior[bb, osl])

    issue_in(0, 0)

    def outer(c0, carry):
        for b in range(2):
            c = c0 * 2 + b
            nb = 1 - b

            @pl.when(c + 1 < NCHUNK)
            def _():
                issue_in(c + 1, nb)

            wait_in(c, b)

            @pl.when(c >= 2)
            def _():
                out_copy(c - 2, b).wait()

            compute_chunk(c, b)
            out_copy(c, b).start()
        return carry

    lax.fori_loop(0, NCHUNK // 2, outer, 0, unroll=False)
    out_copy(NCHUNK - 2, 0).wait()
    out_copy(NCHUNK - 1, 1).wait()


@jax.jit
def _sc_call(cond_flat, maskx, priors, unconditionals, conditionals):
    mesh = plsc.VectorSubcoreMesh(core_axis_name="c", subcore_axis_name="s")
    kfn = pl.kernel(
        _sc_body,
        mesh=mesh,
        out_type=jax.ShapeDtypeStruct((B, FLAT), jnp.float32),
        scratch_types=[
            pltpu.VMEM((NODES_PER_W * B,), jnp.int32),
            pltpu.VMEM((NODES_PER_W * B * LANES,), jnp.float32),
            pltpu.VMEM((NUM_RULES,), jnp.float32),
            pltpu.VMEM((2, PAIRS, NUM_RULES), jnp.float32),
            pltpu.VMEM((2, B, ECHUNK), jnp.float32),
            pltpu.VMEM((2, B, ECHUNK), jnp.float32),
            pltpu.SemaphoreType.DMA,
            pltpu.SemaphoreType.DMA,
            pltpu.SemaphoreType.DMA,
            pltpu.SemaphoreType.DMA,
            pltpu.SemaphoreType.DMA,
            pltpu.SemaphoreType.DMA,
        ],
    )
    return kfn(cond_flat, maskx, priors, unconditionals, conditionals)


def kernel(cond_inds, node_mask, full_logit_priors, unconditionals, conditionals):
    cond_flat = cond_inds.T.reshape(-1)
    maskx = jnp.broadcast_to(
        node_mask.T.astype(jnp.float32).reshape(B * MAX_NODES, 1),
        (B * MAX_NODES, LANES)).reshape(-1)
    out = _sc_call(cond_flat, maskx, full_logit_priors, unconditionals,
                   conditionals)
    return out, full_logit_priors
